# Initial kernel scaffold; baseline (speedup 1.0000x reference)
import jax
import jax.numpy as jnp
from jax.experimental import pallas as pl


def kernel(x, edge_index, edge_type, edge_attr, w1, q1, k1, le1, e1, b1, w2, q2, k2, le2, e2, b2):
    raise NotImplementedError("write your pallas kernel here")


# speedup vs baseline: 29.5381x; 29.5381x over previous
import functools

import jax
import jax.numpy as jnp
from jax import lax
from jax.experimental import pallas as pl
from jax.experimental.pallas import tpu as pltpu
from jax.experimental.pallas import tpu_sc as plsc

F = 128
R = 3
NEG_SLOPE = 0.2
EPS = 1e-16

NC = 2
NS = 16
NW = NC * NS
L = 16


def _edge_preprocess(src2, dst2, typ2, ea2, le1, e1, le2, e2, n):
    er, w = src2.shape
    br = er
    grid = (er // br,)
    blk = lambda e: (e, 0)

    def body(src_ref, dst_ref, typ_ref, ea_ref, le1_ref, e1_ref,
             le2_ref, e2_ref, iq_ref, ik_ref, ae1_ref, ae2_ref):
        typ = typ_ref[...]
        iq_ref[...] = typ * n + dst_ref[...]
        ik_ref[...] = typ * n + src_ref[...]
        ea = ea_ref[...]
        c1 = jnp.sum(le1_ref[...] * e1_ref[...])
        c2 = jnp.sum(le2_ref[...] * e2_ref[...])
        ae1_ref[...] = ea * c1
        ae2_ref[...] = ea * c2

    return pl.pallas_call(
        body,
        grid=grid,
        in_specs=[
            pl.BlockSpec((br, w), blk),
            pl.BlockSpec((br, w), blk),
            pl.BlockSpec((br, w), blk),
            pl.BlockSpec((br, w), blk),
            pl.BlockSpec((1, F), lambda e: (0, 0)),
            pl.BlockSpec((1, F), lambda e: (0, 0)),
            pl.BlockSpec((1, F), lambda e: (0, 0)),
            pl.BlockSpec((1, F), lambda e: (0, 0)),
        ],
        out_specs=[
            pl.BlockSpec((br, w), blk),
            pl.BlockSpec((br, w), blk),
            pl.BlockSpec((br, w), blk),
            pl.BlockSpec((br, w), blk),
        ],
        out_shape=[
            jax.ShapeDtypeStruct((er, w), jnp.int32),
            jax.ShapeDtypeStruct((er, w), jnp.int32),
            jax.ShapeDtypeStruct((er, w), jnp.float32),
            jax.ShapeDtypeStruct((er, w), jnp.float32),
        ],
    )(src2, dst2, typ2, ea2, le1, e1, le2, e2)


def _dense1_body(x_ref, w_ref, q_ref, k_ref, h_ref, qn_ref, kn_ref):
    h = lax.dot_general(x_ref[...], w_ref[0], (((1,), (0,)), ((), ())),
                        precision=lax.Precision.HIGHEST,
                        preferred_element_type=jnp.float32)
    h_ref[...] = h
    bn = h.shape[0]
    qn_ref[...] = jnp.broadcast_to(
        jnp.sum(h * q_ref[...], axis=1, keepdims=True), (bn, L))
    kn_ref[...] = jnp.broadcast_to(
        jnp.sum(h * k_ref[...], axis=1, keepdims=True), (bn, L))


def _dense1(x, w, qt, kt):
    n = x.shape[0]
    bn = 1000 if n % 1000 == 0 else n
    nb = n // bn
    return pl.pallas_call(
        _dense1_body,
        grid=(R, nb),
        in_specs=[
            pl.BlockSpec((bn, F), lambda r, b: (b, 0)),
            pl.BlockSpec((1, F, F), lambda r, b: (r, 0, 0)),
            pl.BlockSpec((1, F), lambda r, b: (0, 0)),
            pl.BlockSpec((1, F), lambda r, b: (0, 0)),
        ],
        out_specs=[
            pl.BlockSpec((bn, F), lambda r, b: (r * nb + b, 0)),
            pl.BlockSpec((bn, L), lambda r, b: (r * nb + b, 0)),
            pl.BlockSpec((bn, L), lambda r, b: (r * nb + b, 0)),
        ],
        out_shape=[
            jax.ShapeDtypeStruct((R * n, F), jnp.float32),
            jax.ShapeDtypeStruct((R * n, L), jnp.float32),
            jax.ShapeDtypeStruct((R * n, L), jnp.float32),
        ],
    )(x, w, qt, kt)


def _dense2_body(s_ref, d_ref, b_ref, w_ref, q_ref, k_ref,
                 h_ref, qn_ref, kn_ref):
    s = s_ref[0] + s_ref[1]
    d = d_ref[0, :, 0:1] + d_ref[1, :, 0:1]
    hin = jnp.maximum(s / (d + EPS) + b_ref[...], 0.0)
    h = lax.dot_general(hin, w_ref[0], (((1,), (0,)), ((), ())),
                        precision=lax.Precision.HIGHEST,
                        preferred_element_type=jnp.float32)
    h_ref[...] = h
    bn = h.shape[0]
    qn_ref[...] = jnp.broadcast_to(
        jnp.sum(h * q_ref[...], axis=1, keepdims=True), (bn, L))
    kn_ref[...] = jnp.broadcast_to(
        jnp.sum(h * k_ref[...], axis=1, keepdims=True), (bn, L))


def _dense2(s, d, bvec, w, qt, kt):
    n = s.shape[1]
    bn = 1000 if n % 1000 == 0 else n
    nb = n // bn
    return pl.pallas_call(
        _dense2_body,
        grid=(R, nb),
        in_specs=[
            pl.BlockSpec((NC, bn, F), lambda r, b: (0, b, 0)),
            pl.BlockSpec((NC, bn, L), lambda r, b: (0, b, 0)),
            pl.BlockSpec((1, F), lambda r, b: (0, 0)),
            pl.BlockSpec((1, F, F), lambda r, b: (r, 0, 0)),
            pl.BlockSpec((1, F), lambda r, b: (0, 0)),
            pl.BlockSpec((1, F), lambda r, b: (0, 0)),
        ],
        out_specs=[
            pl.BlockSpec((bn, F), lambda r, b: (r * nb + b, 0)),
            pl.BlockSpec((bn, L), lambda r, b: (r * nb + b, 0)),
            pl.BlockSpec((bn, L), lambda r, b: (r * nb + b, 0)),
        ],
        out_shape=[
            jax.ShapeDtypeStruct((R * n, F), jnp.float32),
            jax.ShapeDtypeStruct((R * n, L), jnp.float32),
            jax.ShapeDtypeStruct((R * n, L), jnp.float32),
        ],
    )(s, d, bvec, w, qt, kt)


def _finish_body(s_ref, d_ref, b_ref, out_ref):
    s = s_ref[0] + s_ref[1]
    d = d_ref[0, :, 0:1] + d_ref[1, :, 0:1]
    out_ref[...] = s / (d + EPS) + b_ref[...]


def _finish(s, d, bvec):
    n = s.shape[1]
    bn = 1000 if n % 1000 == 0 else n
    nb = n // bn
    return pl.pallas_call(
        _finish_body,
        grid=(nb,),
        in_specs=[
            pl.BlockSpec((NC, bn, F), lambda b: (0, b, 0)),
            pl.BlockSpec((NC, bn, L), lambda b: (0, b, 0)),
            pl.BlockSpec((1, F), lambda b: (0, 0)),
        ],
        out_specs=pl.BlockSpec((bn, F), lambda b: (b, 0)),
        out_shape=jax.ShapeDtypeStruct((n, F), jnp.float32),
    )(s, d, bvec)


def _make_sc_layer(n, e, interpret=False):
    ew = e // NW
    assert ew * NW == e
    ch = 80
    while ew % ch:
        ch -= 16
    nchunk = ew // ch
    nzc = n // ch
    assert nzc * ch == n
    zrounds = (nzc + NS - 1) // NS
    rn = R * n

    mesh = plsc.VectorSubcoreMesh(core_axis_name="c", subcore_axis_name="s",
                                  num_cores=NC, num_subcores=NS)

    @functools.partial(
        pl.kernel,
        out_type=[
            jax.ShapeDtypeStruct((NC, n, F), jnp.float32),
            jax.ShapeDtypeStruct((NC, n, L), jnp.float32),
        ],
        mesh=mesh,
        interpret=interpret,
        compiler_params=pltpu.CompilerParams(needs_layout_passes=False,
                                             use_tc_tiling_on_sc=False),
        scratch_types=[
            pltpu.VMEM((ch, L), jnp.float32),
            pltpu.VMEM((ch, L), jnp.float32),
            pltpu.VMEM((ch,), jnp.int32),
            pltpu.VMEM((ch,), jnp.int32),
            pltpu.VMEM((ch,), jnp.int32),
            pltpu.VMEM((ch,), jnp.float32),
            pltpu.VMEM((ch, L), jnp.float32),
            pltpu.VMEM((ch, F), jnp.float32),
            pltpu.VMEM_SHARED((n, F), jnp.float32),
            pltpu.VMEM_SHARED((n, L), jnp.float32),
            pltpu.SemaphoreType.DMA,
            pltpu.SemaphoreType.DMA,
            pltpu.SemaphoreType.DMA,
        ],
    )
    def sc_layer(h_hbm, qn_hbm, kn_hbm, iq_hbm, ik_hbm, dst_hbm, ae_hbm,
                 s_out, d_out, qv_ch, kv_ch, iq_v, ik_v, dst_v, ae_v,
                 exrow, rows, s_sh, d_sh, gsem, qsem, ksem):
        cid = lax.axis_index("c")
        sid = lax.axis_index("s")
        wid = cid * NS + sid

        zf = jnp.zeros((L,), jnp.float32)

        def zrows(i, carry):
            for j in range(F // L):
                rows[i, pl.ds(j * L, L)] = zf
            exrow[i, :] = zf
            return carry

        lax.fori_loop(0, ch, zrows, 0)

        def zchunk(m, carry):
            c = sid + NS * m

            @pl.when(c < nzc)
            def _():
                pltpu.sync_copy(rows, s_sh.at[pl.ds(c * ch, ch)])
                pltpu.sync_copy(exrow, d_sh.at[pl.ds(c * ch, ch)])

            return carry

        lax.fori_loop(0, zrounds, zchunk, 0)
        plsc.subcore_barrier()

        def chunk_body(t, carry):
            base = wid * ew + t * ch
            pltpu.sync_copy(ik_hbm.at[pl.ds(base, ch)], ik_v)
            pltpu.sync_copy(iq_hbm.at[pl.ds(base, ch)], iq_v)
            cp = pltpu.async_copy(h_hbm.at[ik_v], rows, gsem)
            cq = pltpu.async_copy(qn_hbm.at[iq_v], qv_ch, qsem)
            ck = pltpu.async_copy(kn_hbm.at[ik_v], kv_ch, ksem)
            pltpu.sync_copy(dst_hbm.at[pl.ds(base, ch)], dst_v)
            pltpu.sync_copy(ae_hbm.at[pl.ds(base, ch)], ae_v)
            cq.wait()
            ck.wait()
            cp.wait()

            def edge(i, c2):
                sp_ae = plsc.load_gather(ae_v, [jnp.zeros((L,), jnp.int32) + i])
                al = qv_ch[i, :] + kv_ch[i, :] + sp_ae
                al = jnp.maximum(al, NEG_SLOPE * al)
                ex = jnp.exp(al)
                exrow[i, :] = ex
                for j in range(F // L):
                    rows[i, pl.ds(j * L, L)] = rows[i, pl.ds(j * L, L)] * ex
                return c2

            lax.fori_loop(0, ch, edge, 0)
            pltpu.sync_copy(rows, s_sh.at[dst_v], add=True)
            pltpu.sync_copy(exrow, d_sh.at[dst_v], add=True)
            return carry

        lax.fori_loop(0, nchunk, chunk_body, 0)
        plsc.subcore_barrier()

        def ochunk(m, carry):
            c = sid + NS * m

            @pl.when(c < nzc)
            def _():
                pltpu.sync_copy(s_sh.at[pl.ds(c * ch, ch)],
                                s_out.at[cid].at[pl.ds(c * ch, ch)])
                pltpu.sync_copy(d_sh.at[pl.ds(c * ch, ch)],
                                d_out.at[cid].at[pl.ds(c * ch, ch)])

            return carry

        lax.fori_loop(0, zrounds, ochunk, 0)

    return sc_layer


def kernel(x, edge_index, edge_type, edge_attr,
           w1, q1, k1, le1, e1, b1,
           w2, q2, k2, le2, e2, b2):
    n = x.shape[0]
    e = edge_index.shape[1]
    er = e // F
    assert er * F == e

    src = edge_index[0]
    dst = edge_index[1]
    le1t = le1.reshape(1, F)
    e1t = e1.reshape(1, F)
    le2t = le2.reshape(1, F)
    e2t = e2.reshape(1, F)

    iq2, ik2, ae1_2, ae2_2 = _edge_preprocess(
        src.reshape(er, F), dst.reshape(er, F),
        edge_type.reshape(er, F), edge_attr.reshape(er, F),
        le1t, e1t, le2t, e2t, n)
    iq = iq2.reshape(e)
    ik = ik2.reshape(e)
    ae1 = ae1_2.reshape(e)
    ae2 = ae2_2.reshape(e)

    sc_layer = _make_sc_layer(n, e)

    h1, qn1, kn1 = _dense1(x, w1, q1.reshape(1, F), k1.reshape(1, F))
    s1, d1 = sc_layer(h1, qn1, kn1, iq, ik, dst, ae1)
    h2, qn2, kn2 = _dense2(s1, d1, b1.reshape(1, F), w2,
                           q2.reshape(1, F), k2.reshape(1, F))
    s2, d2 = sc_layer(h2, qn2, kn2, iq, ik, dst, ae2)
    return _finish(s2, d2, b2.reshape(1, F))


# speedup vs baseline: 38.6793x; 1.3095x over previous
import functools

import jax
import jax.numpy as jnp
from jax import lax
from jax.experimental import pallas as pl
from jax.experimental.pallas import tpu as pltpu
from jax.experimental.pallas import tpu_sc as plsc

F = 128
R = 3
NEG_SLOPE = 0.2
EPS = 1e-16

NC = 2
NS = 16
NW = NC * NS
L = 16

_SPLAT_DN = lax.GatherDimensionNumbers(offset_dims=(),
                                       collapsed_slice_dims=(0,),
                                       start_index_map=(0,))


def _splat_lane(x, idx):
    return lax.gather(x, idx[:, None], _SPLAT_DN, slice_sizes=(1,),
                      mode=lax.GatherScatterMode.PROMISE_IN_BOUNDS)


def _edge_preprocess(src2, dst2, typ2, ea2, le1, e1, le2, e2, n):
    er, w = src2.shape
    br = er
    grid = (er // br,)
    blk = lambda e: (e, 0)

    def body(src_ref, dst_ref, typ_ref, ea_ref, le1_ref, e1_ref,
             le2_ref, e2_ref, iq_ref, ik_ref, ae1_ref, ae2_ref):
        typ = typ_ref[...]
        iq_ref[...] = typ * n + dst_ref[...]
        ik_ref[...] = typ * n + src_ref[...]
        ea = ea_ref[...]
        c1 = jnp.sum(le1_ref[...] * e1_ref[...])
        c2 = jnp.sum(le2_ref[...] * e2_ref[...])
        ae1_ref[...] = ea * c1
        ae2_ref[...] = ea * c2

    return pl.pallas_call(
        body,
        grid=grid,
        in_specs=[
            pl.BlockSpec((br, w), blk),
            pl.BlockSpec((br, w), blk),
            pl.BlockSpec((br, w), blk),
            pl.BlockSpec((br, w), blk),
            pl.BlockSpec((1, F), lambda e: (0, 0)),
            pl.BlockSpec((1, F), lambda e: (0, 0)),
            pl.BlockSpec((1, F), lambda e: (0, 0)),
            pl.BlockSpec((1, F), lambda e: (0, 0)),
        ],
        out_specs=[
            pl.BlockSpec((br, w), blk),
            pl.BlockSpec((br, w), blk),
            pl.BlockSpec((br, w), blk),
            pl.BlockSpec((br, w), blk),
        ],
        out_shape=[
            jax.ShapeDtypeStruct((er, w), jnp.int32),
            jax.ShapeDtypeStruct((er, w), jnp.int32),
            jax.ShapeDtypeStruct((er, w), jnp.float32),
            jax.ShapeDtypeStruct((er, w), jnp.float32),
        ],
    )(src2, dst2, typ2, ea2, le1, e1, le2, e2)


def _dense1_body(x_ref, w_ref, q_ref, k_ref, h_ref, qn_ref, kn_ref):
    h = lax.dot_general(x_ref[...], w_ref[0], (((1,), (0,)), ((), ())),
                        precision=lax.Precision.HIGHEST,
                        preferred_element_type=jnp.float32)
    h_ref[...] = h
    bn = h.shape[0]
    qn_ref[...] = jnp.broadcast_to(
        jnp.sum(h * q_ref[...], axis=1, keepdims=True), (bn, L))
    kn_ref[...] = jnp.broadcast_to(
        jnp.sum(h * k_ref[...], axis=1, keepdims=True), (bn, L))


def _dense1(x, w, qt, kt):
    n = x.shape[0]
    bn = 1000 if n % 1000 == 0 else n
    nb = n // bn
    return pl.pallas_call(
        _dense1_body,
        grid=(R, nb),
        in_specs=[
            pl.BlockSpec((bn, F), lambda r, b: (b, 0)),
            pl.BlockSpec((1, F, F), lambda r, b: (r, 0, 0)),
            pl.BlockSpec((1, F), lambda r, b: (0, 0)),
            pl.BlockSpec((1, F), lambda r, b: (0, 0)),
        ],
        out_specs=[
            pl.BlockSpec((bn, F), lambda r, b: (r * nb + b, 0)),
            pl.BlockSpec((bn, L), lambda r, b: (r * nb + b, 0)),
            pl.BlockSpec((bn, L), lambda r, b: (r * nb + b, 0)),
        ],
        out_shape=[
            jax.ShapeDtypeStruct((R * n, F), jnp.float32),
            jax.ShapeDtypeStruct((R * n, L), jnp.float32),
            jax.ShapeDtypeStruct((R * n, L), jnp.float32),
        ],
    )(x, w, qt, kt)


def _dense2_body(s_ref, d_ref, b_ref, w_ref, q_ref, k_ref,
                 h_ref, qn_ref, kn_ref):
    s = s_ref[0] + s_ref[1]
    d = d_ref[0, :, 0:1] + d_ref[1, :, 0:1]
    hin = jnp.maximum(s / (d + EPS) + b_ref[...], 0.0)
    h = lax.dot_general(hin, w_ref[0], (((1,), (0,)), ((), ())),
                        precision=lax.Precision.HIGHEST,
                        preferred_element_type=jnp.float32)
    h_ref[...] = h
    bn = h.shape[0]
    qn_ref[...] = jnp.broadcast_to(
        jnp.sum(h * q_ref[...], axis=1, keepdims=True), (bn, L))
    kn_ref[...] = jnp.broadcast_to(
        jnp.sum(h * k_ref[...], axis=1, keepdims=True), (bn, L))


def _dense2(s, d, bvec, w, qt, kt):
    n = s.shape[1]
    bn = 1000 if n % 1000 == 0 else n
    nb = n // bn
    return pl.pallas_call(
        _dense2_body,
        grid=(R, nb),
        in_specs=[
            pl.BlockSpec((NC, bn, F), lambda r, b: (0, b, 0)),
            pl.BlockSpec((NC, bn, L), lambda r, b: (0, b, 0)),
            pl.BlockSpec((1, F), lambda r, b: (0, 0)),
            pl.BlockSpec((1, F, F), lambda r, b: (r, 0, 0)),
            pl.BlockSpec((1, F), lambda r, b: (0, 0)),
            pl.BlockSpec((1, F), lambda r, b: (0, 0)),
        ],
        out_specs=[
            pl.BlockSpec((bn, F), lambda r, b: (r * nb + b, 0)),
            pl.BlockSpec((bn, L), lambda r, b: (r * nb + b, 0)),
            pl.BlockSpec((bn, L), lambda r, b: (r * nb + b, 0)),
        ],
        out_shape=[
            jax.ShapeDtypeStruct((R * n, F), jnp.float32),
            jax.ShapeDtypeStruct((R * n, L), jnp.float32),
            jax.ShapeDtypeStruct((R * n, L), jnp.float32),
        ],
    )(s, d, bvec, w, qt, kt)


def _finish_body(s_ref, d_ref, b_ref, out_ref):
    s = s_ref[0] + s_ref[1]
    d = d_ref[0, :, 0:1] + d_ref[1, :, 0:1]
    out_ref[...] = s / (d + EPS) + b_ref[...]


def _finish(s, d, bvec):
    n = s.shape[1]
    bn = 1000 if n % 1000 == 0 else n
    nb = n // bn
    return pl.pallas_call(
        _finish_body,
        grid=(nb,),
        in_specs=[
            pl.BlockSpec((NC, bn, F), lambda b: (0, b, 0)),
            pl.BlockSpec((NC, bn, L), lambda b: (0, b, 0)),
            pl.BlockSpec((1, F), lambda b: (0, 0)),
        ],
        out_specs=pl.BlockSpec((bn, F), lambda b: (b, 0)),
        out_shape=jax.ShapeDtypeStruct((n, F), jnp.float32),
    )(s, d, bvec)


def _make_sc_layer(n, e, interpret=False):
    ew = e // NW
    assert ew * NW == e
    ch = 80
    while ew % ch:
        ch -= 16
    nchunk = ew // ch
    nzc = n // ch
    assert nzc * ch == n
    zrounds = (nzc + NS - 1) // NS

    mesh = plsc.VectorSubcoreMesh(core_axis_name="c", subcore_axis_name="s",
                                  num_cores=NC, num_subcores=NS)

    @functools.partial(
        pl.kernel,
        out_type=[
            jax.ShapeDtypeStruct((NC, n, F), jnp.float32),
            jax.ShapeDtypeStruct((NC, n, L), jnp.float32),
        ],
        mesh=mesh,
        interpret=interpret,
        compiler_params=pltpu.CompilerParams(needs_layout_passes=False,
                                             use_tc_tiling_on_sc=False),
        scratch_types=[
            pltpu.VMEM((ch, L), jnp.float32),
            pltpu.VMEM((ch, L), jnp.float32),
            pltpu.VMEM((ch,), jnp.int32),
            pltpu.VMEM((ch,), jnp.int32),
            pltpu.VMEM((ch,), jnp.int32),
            pltpu.VMEM((ch,), jnp.float32),
            pltpu.VMEM((ch, L), jnp.float32),
            pltpu.VMEM((ch, F), jnp.float32),
            pltpu.VMEM_SHARED((n, F), jnp.float32),
            pltpu.VMEM_SHARED((n, L), jnp.float32),
            pltpu.SemaphoreType.DMA,
            pltpu.SemaphoreType.DMA,
            pltpu.SemaphoreType.DMA,
        ],
    )
    def sc_layer(h_hbm, qn_hbm, kn_hbm, iq_hbm, ik_hbm, dst_hbm, ae_hbm,
                 s_out, d_out, qv_ch, kv_ch, iq_v, ik_v, dst_v, ae_v,
                 exrow, rows, s_sh, d_sh, gsem, qsem, ksem):
        cid = lax.axis_index("c")
        sid = lax.axis_index("s")
        wid = cid * NS + sid

        zf = jnp.zeros((L,), jnp.float32)

        def zrows(i, carry):
            for j in range(F // L):
                rows[i, pl.ds(j * L, L)] = zf
            exrow[i, :] = zf
            return carry

        lax.fori_loop(0, ch, zrows, 0)

        def zchunk(m, carry):
            c = sid + NS * m

            @pl.when(c < nzc)
            def _():
                pltpu.sync_copy(rows, s_sh.at[pl.ds(c * ch, ch)])
                pltpu.sync_copy(exrow, d_sh.at[pl.ds(c * ch, ch)])

            return carry

        lax.fori_loop(0, zrounds, zchunk, 0)
        plsc.subcore_barrier()

        lane = lax.iota(jnp.int32, 16)
        z16 = jnp.zeros((16,), jnp.int32)

        def chunk_body(t, carry):
            base = wid * ew + t * ch
            pltpu.sync_copy(ik_hbm.at[pl.ds(base, ch)], ik_v)
            pltpu.sync_copy(iq_hbm.at[pl.ds(base, ch)], iq_v)
            cp = pltpu.async_copy(h_hbm.at[ik_v], rows, gsem)
            cq = pltpu.async_copy(qn_hbm.at[iq_v], qv_ch, qsem)
            ck = pltpu.async_copy(kn_hbm.at[ik_v], kv_ch, ksem)
            pltpu.sync_copy(dst_hbm.at[pl.ds(base, ch)], dst_v)
            pltpu.sync_copy(ae_hbm.at[pl.ds(base, ch)], ae_v)
            cq.wait()
            ck.wait()
            cp.wait()

            for g in range(ch // 16):
                b16 = g * 16
                idx = lane + b16
                q16 = plsc.load_gather(qv_ch, [idx, z16])
                k16 = plsc.load_gather(kv_ch, [idx, z16])
                a16 = ae_v[pl.ds(b16, 16)]
                al = q16 + k16 + a16
                al = jnp.maximum(al, NEG_SLOPE * al)
                ex16 = jnp.exp(al)
                for ii in range(16):
                    i = b16 + ii
                    sp = _splat_lane(ex16, z16 + ii)
                    exrow[i, :] = sp
                    for j in range(F // L):
                        rows[i, pl.ds(j * L, L)] = rows[i, pl.ds(j * L, L)] * sp

            pltpu.sync_copy(rows, s_sh.at[dst_v], add=True)
            pltpu.sync_copy(exrow, d_sh.at[dst_v], add=True)
            return carry

        lax.fori_loop(0, nchunk, chunk_body, 0)
        plsc.subcore_barrier()

        def ochunk(m, carry):
            c = sid + NS * m

            @pl.when(c < nzc)
            def _():
                pltpu.sync_copy(s_sh.at[pl.ds(c * ch, ch)],
                                s_out.at[cid].at[pl.ds(c * ch, ch)])
                pltpu.sync_copy(d_sh.at[pl.ds(c * ch, ch)],
                                d_out.at[cid].at[pl.ds(c * ch, ch)])

            return carry

        lax.fori_loop(0, zrounds, ochunk, 0)

    return sc_layer


def kernel(x, edge_index, edge_type, edge_attr,
           w1, q1, k1, le1, e1, b1,
           w2, q2, k2, le2, e2, b2):
    n = x.shape[0]
    e = edge_index.shape[1]
    er = e // F
    assert er * F == e

    src = edge_index[0]
    dst = edge_index[1]
    le1t = le1.reshape(1, F)
    e1t = e1.reshape(1, F)
    le2t = le2.reshape(1, F)
    e2t = e2.reshape(1, F)

    iq2, ik2, ae1_2, ae2_2 = _edge_preprocess(
        src.reshape(er, F), dst.reshape(er, F),
        edge_type.reshape(er, F), edge_attr.reshape(er, F),
        le1t, e1t, le2t, e2t, n)
    iq = iq2.reshape(e)
    ik = ik2.reshape(e)
    ae1 = ae1_2.reshape(e)
    ae2 = ae2_2.reshape(e)

    sc_layer = _make_sc_layer(n, e)

    h1, qn1, kn1 = _dense1(x, w1, q1.reshape(1, F), k1.reshape(1, F))
    s1, d1 = sc_layer(h1, qn1, kn1, iq, ik, dst, ae1)
    h2, qn2, kn2 = _dense2(s1, d1, b1.reshape(1, F), w2,
                           q2.reshape(1, F), k2.reshape(1, F))
    s2, d2 = sc_layer(h2, qn2, kn2, iq, ik, dst, ae2)
    return _finish(s2, d2, b2.reshape(1, F))


# speedup vs baseline: 46.0369x; 1.1902x over previous
import functools

import jax
import jax.numpy as jnp
from jax import lax
from jax.experimental import pallas as pl
from jax.experimental.pallas import tpu as pltpu
from jax.experimental.pallas import tpu_sc as plsc

F = 128
R = 3
NEG_SLOPE = 0.2
EPS = 1e-16

NC = 2
NS = 16
NW = NC * NS
L = 16

_SPLAT_DN = lax.GatherDimensionNumbers(offset_dims=(),
                                       collapsed_slice_dims=(0,),
                                       start_index_map=(0,))


def _splat_lane(x, idx):
    return lax.gather(x, idx[:, None], _SPLAT_DN, slice_sizes=(1,),
                      mode=lax.GatherScatterMode.PROMISE_IN_BOUNDS)


def _edge_preprocess(src2, dst2, typ2, ea2, le1, e1, le2, e2, n):
    er, w = src2.shape
    br = er
    grid = (er // br,)
    blk = lambda e: (e, 0)

    def body(src_ref, dst_ref, typ_ref, ea_ref, le1_ref, e1_ref,
             le2_ref, e2_ref, iq_ref, ik_ref, ae1_ref, ae2_ref):
        typ = typ_ref[...]
        iq_ref[...] = typ * n + dst_ref[...]
        ik_ref[...] = typ * n + src_ref[...]
        ea = ea_ref[...]
        c1 = jnp.sum(le1_ref[...] * e1_ref[...])
        c2 = jnp.sum(le2_ref[...] * e2_ref[...])
        ae1_ref[...] = ea * c1
        ae2_ref[...] = ea * c2

    return pl.pallas_call(
        body,
        grid=grid,
        in_specs=[
            pl.BlockSpec((br, w), blk),
            pl.BlockSpec((br, w), blk),
            pl.BlockSpec((br, w), blk),
            pl.BlockSpec((br, w), blk),
            pl.BlockSpec((1, F), lambda e: (0, 0)),
            pl.BlockSpec((1, F), lambda e: (0, 0)),
            pl.BlockSpec((1, F), lambda e: (0, 0)),
            pl.BlockSpec((1, F), lambda e: (0, 0)),
        ],
        out_specs=[
            pl.BlockSpec((br, w), blk),
            pl.BlockSpec((br, w), blk),
            pl.BlockSpec((br, w), blk),
            pl.BlockSpec((br, w), blk),
        ],
        out_shape=[
            jax.ShapeDtypeStruct((er, w), jnp.int32),
            jax.ShapeDtypeStruct((er, w), jnp.int32),
            jax.ShapeDtypeStruct((er, w), jnp.float32),
            jax.ShapeDtypeStruct((er, w), jnp.float32),
        ],
    )(src2, dst2, typ2, ea2, le1, e1, le2, e2)


def _dense1_body(x_ref, w_ref, q_ref, k_ref, h_ref, qn_ref, kn_ref):
    h = lax.dot_general(x_ref[...], w_ref[0], (((1,), (0,)), ((), ())),
                        precision=lax.Precision.HIGHEST,
                        preferred_element_type=jnp.float32)
    h_ref[...] = h
    bn = h.shape[0]
    qn_ref[...] = jnp.broadcast_to(
        jnp.sum(h * q_ref[...], axis=1, keepdims=True), (bn, L))
    kn_ref[...] = jnp.broadcast_to(
        jnp.sum(h * k_ref[...], axis=1, keepdims=True), (bn, L))


def _dense1(x, w, qt, kt):
    n = x.shape[0]
    bn = 1000 if n % 1000 == 0 else n
    nb = n // bn
    return pl.pallas_call(
        _dense1_body,
        grid=(R, nb),
        in_specs=[
            pl.BlockSpec((bn, F), lambda r, b: (b, 0)),
            pl.BlockSpec((1, F, F), lambda r, b: (r, 0, 0)),
            pl.BlockSpec((1, F), lambda r, b: (0, 0)),
            pl.BlockSpec((1, F), lambda r, b: (0, 0)),
        ],
        out_specs=[
            pl.BlockSpec((bn, F), lambda r, b: (r * nb + b, 0)),
            pl.BlockSpec((bn, L), lambda r, b: (r * nb + b, 0)),
            pl.BlockSpec((bn, L), lambda r, b: (r * nb + b, 0)),
        ],
        out_shape=[
            jax.ShapeDtypeStruct((R * n, F), jnp.float32),
            jax.ShapeDtypeStruct((R * n, L), jnp.float32),
            jax.ShapeDtypeStruct((R * n, L), jnp.float32),
        ],
    )(x, w, qt, kt)


def _dense2_body(s_ref, d_ref, b_ref, w_ref, q_ref, k_ref,
                 h_ref, qn_ref, kn_ref):
    s = s_ref[0] + s_ref[1]
    d = d_ref[0, :, 0:1] + d_ref[1, :, 0:1]
    hin = jnp.maximum(s / (d + EPS) + b_ref[...], 0.0)
    h = lax.dot_general(hin, w_ref[0], (((1,), (0,)), ((), ())),
                        precision=lax.Precision.HIGHEST,
                        preferred_element_type=jnp.float32)
    h_ref[...] = h
    bn = h.shape[0]
    qn_ref[...] = jnp.broadcast_to(
        jnp.sum(h * q_ref[...], axis=1, keepdims=True), (bn, L))
    kn_ref[...] = jnp.broadcast_to(
        jnp.sum(h * k_ref[...], axis=1, keepdims=True), (bn, L))


def _dense2(s, d, bvec, w, qt, kt):
    n = s.shape[1]
    bn = 1000 if n % 1000 == 0 else n
    nb = n // bn
    return pl.pallas_call(
        _dense2_body,
        grid=(R, nb),
        in_specs=[
            pl.BlockSpec((NC, bn, F), lambda r, b: (0, b, 0)),
            pl.BlockSpec((NC, bn, L), lambda r, b: (0, b, 0)),
            pl.BlockSpec((1, F), lambda r, b: (0, 0)),
            pl.BlockSpec((1, F, F), lambda r, b: (r, 0, 0)),
            pl.BlockSpec((1, F), lambda r, b: (0, 0)),
            pl.BlockSpec((1, F), lambda r, b: (0, 0)),
        ],
        out_specs=[
            pl.BlockSpec((bn, F), lambda r, b: (r * nb + b, 0)),
            pl.BlockSpec((bn, L), lambda r, b: (r * nb + b, 0)),
            pl.BlockSpec((bn, L), lambda r, b: (r * nb + b, 0)),
        ],
        out_shape=[
            jax.ShapeDtypeStruct((R * n, F), jnp.float32),
            jax.ShapeDtypeStruct((R * n, L), jnp.float32),
            jax.ShapeDtypeStruct((R * n, L), jnp.float32),
        ],
    )(s, d, bvec, w, qt, kt)


def _finish_body(s_ref, d_ref, b_ref, out_ref):
    s = s_ref[0] + s_ref[1]
    d = d_ref[0, :, 0:1] + d_ref[1, :, 0:1]
    out_ref[...] = s / (d + EPS) + b_ref[...]


def _finish(s, d, bvec):
    n = s.shape[1]
    bn = 1000 if n % 1000 == 0 else n
    nb = n // bn
    return pl.pallas_call(
        _finish_body,
        grid=(nb,),
        in_specs=[
            pl.BlockSpec((NC, bn, F), lambda b: (0, b, 0)),
            pl.BlockSpec((NC, bn, L), lambda b: (0, b, 0)),
            pl.BlockSpec((1, F), lambda b: (0, 0)),
        ],
        out_specs=pl.BlockSpec((bn, F), lambda b: (b, 0)),
        out_shape=jax.ShapeDtypeStruct((n, F), jnp.float32),
    )(s, d, bvec)


def _make_sc_layer(n, e, interpret=False):
    ew = e // NW
    assert ew * NW == e
    ch = 80
    while ew % ch:
        ch -= 16
    nchunk = ew // ch
    assert nchunk % 2 == 1 and nchunk >= 3
    npairs = (nchunk - 1) // 2
    nzc = n // ch
    assert nzc * ch == n
    zrounds = (nzc + NS - 1) // NS

    mesh = plsc.VectorSubcoreMesh(core_axis_name="c", subcore_axis_name="s",
                                  num_cores=NC, num_subcores=NS)

    @functools.partial(
        pl.kernel,
        out_type=[
            jax.ShapeDtypeStruct((NC, n, F), jnp.float32),
            jax.ShapeDtypeStruct((NC, n, L), jnp.float32),
        ],
        mesh=mesh,
        interpret=interpret,
        compiler_params=pltpu.CompilerParams(needs_layout_passes=False,
                                             use_tc_tiling_on_sc=False),
        scratch_types=[
            pltpu.VMEM((ch, L), jnp.float32),
            pltpu.VMEM((ch, L), jnp.float32),
            pltpu.VMEM((ch, L), jnp.float32),
            pltpu.VMEM((ch, L), jnp.float32),
            pltpu.VMEM((ch, F), jnp.float32),
            pltpu.VMEM((ch, F), jnp.float32),
            pltpu.VMEM((4 * ch,), jnp.int32),
            pltpu.VMEM((4 * ch,), jnp.int32),
            pltpu.VMEM((ch, L), jnp.float32),
            pltpu.VMEM_SHARED((n, F), jnp.float32),
            pltpu.VMEM_SHARED((n, L), jnp.float32),
            pltpu.SemaphoreType.DMA,
            pltpu.SemaphoreType.DMA,
            pltpu.SemaphoreType.DMA,
            pltpu.SemaphoreType.DMA,
            pltpu.SemaphoreType.DMA,
            pltpu.SemaphoreType.DMA,
        ],
    )
    def sc_layer(h_hbm, qn_hbm, kn_hbm, pk_hbm,
                 s_out, d_out, qvA, qvB, kvA, kvB, rowsA, rowsB,
                 pbufA, pbufB, exrow, s_sh, d_sh,
                 gsA, qsA, ksA, gsB, qsB, ksB):
        cid = lax.axis_index("c")
        sid = lax.axis_index("s")
        wid = cid * NS + sid

        zf = jnp.zeros((L,), jnp.float32)

        def zrows(i, carry):
            for j in range(F // L):
                rowsA[i, pl.ds(j * L, L)] = zf
            exrow[i, :] = zf
            return carry

        lax.fori_loop(0, ch, zrows, 0)

        def zchunk(m, carry):
            c = sid + NS * m

            @pl.when(c < nzc)
            def _():
                pltpu.sync_copy(rowsA, s_sh.at[pl.ds(c * ch, ch)])
                pltpu.sync_copy(exrow, d_sh.at[pl.ds(c * ch, ch)])

            return carry

        lax.fori_loop(0, zrounds, zchunk, 0)
        plsc.subcore_barrier()

        lane = lax.iota(jnp.int32, 16)
        z16 = jnp.zeros((16,), jnp.int32)

        def idx_load(t, pbuf):
            g = wid * nchunk + t
            pltpu.sync_copy(pk_hbm.at[pl.ds(g * 4 * ch, 4 * ch)], pbuf)

        def gather_copies(pbuf, rows_x, qv_x, kv_x, gs, qs, ks):
            iq_r = pbuf.at[pl.ds(0, ch)]
            ik_r = pbuf.at[pl.ds(ch, ch)]
            return (
                pltpu.make_async_copy(h_hbm.at[ik_r], rows_x, gs),
                pltpu.make_async_copy(qn_hbm.at[iq_r], qv_x, qs),
                pltpu.make_async_copy(kn_hbm.at[ik_r], kv_x, ks),
            )

        def issue(bufs):
            for c in gather_copies(*bufs):
                c.start()

        def wait(bufs):
            for c in gather_copies(*bufs):
                c.wait()

        def compute(pbuf, rows_x, qv_x, kv_x, *_):
            for g5 in range(ch // 16):
                b16 = g5 * 16
                idx = lane + b16
                q16 = plsc.load_gather(qv_x, [idx, z16])
                k16 = plsc.load_gather(kv_x, [idx, z16])
                a16 = plsc.bitcast(pbuf[pl.ds(3 * ch + b16, 16)], jnp.float32)
                al = q16 + k16 + a16
                al = jnp.maximum(al, NEG_SLOPE * al)
                ex16 = jnp.exp(al)
                for ii in range(16):
                    i = b16 + ii
                    sp = _splat_lane(ex16, z16 + ii)
                    exrow[i, :] = sp
                    for j in range(F // L):
                        rows_x[i, pl.ds(j * L, L)] = (
                            rows_x[i, pl.ds(j * L, L)] * sp)

            dst_r = pbuf.at[pl.ds(2 * ch, ch)]
            pltpu.sync_copy(rows_x, s_sh.at[dst_r], add=True)
            pltpu.sync_copy(exrow, d_sh.at[dst_r], add=True)

        bufsA = (pbufA, rowsA, qvA, kvA, gsA, qsA, ksA)
        bufsB = (pbufB, rowsB, qvB, kvB, gsB, qsB, ksB)

        idx_load(0, pbufA)
        issue(bufsA)
        idx_load(1, pbufB)

        def pair(p, carry):
            t0 = 2 * p
            issue(bufsB)
            wait(bufsA)
            compute(*bufsA)
            idx_load(t0 + 2, pbufA)
            issue(bufsA)
            wait(bufsB)
            compute(*bufsB)

            @pl.when(t0 + 3 < nchunk)
            def _():
                idx_load(t0 + 3, pbufB)

            return carry

        lax.fori_loop(0, npairs, pair, 0)
        wait(bufsA)
        compute(*bufsA)
        plsc.subcore_barrier()

        def ochunk(m, carry):
            c = sid + NS * m

            @pl.when(c < nzc)
            def _():
                pltpu.sync_copy(s_sh.at[pl.ds(c * ch, ch)],
                                s_out.at[cid].at[pl.ds(c * ch, ch)])
                pltpu.sync_copy(d_sh.at[pl.ds(c * ch, ch)],
                                d_out.at[cid].at[pl.ds(c * ch, ch)])

            return carry

        lax.fori_loop(0, zrounds, ochunk, 0)

    return sc_layer


def kernel(x, edge_index, edge_type, edge_attr,
           w1, q1, k1, le1, e1, b1,
           w2, q2, k2, le2, e2, b2):
    n = x.shape[0]
    e = edge_index.shape[1]
    er = e // F
    assert er * F == e

    src = edge_index[0]
    dst = edge_index[1]
    le1t = le1.reshape(1, F)
    e1t = e1.reshape(1, F)
    le2t = le2.reshape(1, F)
    e2t = e2.reshape(1, F)

    iq2, ik2, ae1_2, ae2_2 = _edge_preprocess(
        src.reshape(er, F), dst.reshape(er, F),
        edge_type.reshape(er, F), edge_attr.reshape(er, F),
        le1t, e1t, le2t, e2t, n)
    iq = iq2.reshape(e)
    ik = ik2.reshape(e)
    ae1 = ae1_2.reshape(e)
    ae2 = ae2_2.reshape(e)

    ew = e // NW
    ch = 80
    while ew % ch:
        ch -= 16

    def _pack(ae):
        cols = [iq.reshape(-1, ch), ik.reshape(-1, ch), dst.reshape(-1, ch),
                lax.bitcast_convert_type(ae, jnp.int32).reshape(-1, ch)]
        return jnp.stack(cols, axis=1).reshape(-1)

    pk1 = _pack(ae1)
    pk2 = _pack(ae2)

    sc_layer = _make_sc_layer(n, e)

    h1, qn1, kn1 = _dense1(x, w1, q1.reshape(1, F), k1.reshape(1, F))
    s1, d1 = sc_layer(h1, qn1, kn1, pk1)
    h2, qn2, kn2 = _dense2(s1, d1, b1.reshape(1, F), w2,
                           q2.reshape(1, F), k2.reshape(1, F))
    s2, d2 = sc_layer(h2, qn2, kn2, pk2)
    return _finish(s2, d2, b2.reshape(1, F))


# speedup vs baseline: 48.0131x; 1.0429x over previous
import functools

import jax
import jax.numpy as jnp
from jax import lax
from jax.experimental import pallas as pl
from jax.experimental.pallas import tpu as pltpu
from jax.experimental.pallas import tpu_sc as plsc

F = 128
R = 3
NEG_SLOPE = 0.2
EPS = 1e-16

NC = 2
NS = 16
NW = NC * NS
L = 16

_SPLAT_DN = lax.GatherDimensionNumbers(offset_dims=(),
                                       collapsed_slice_dims=(0,),
                                       start_index_map=(0,))


def _splat_lane(x, idx):
    return lax.gather(x, idx[:, None], _SPLAT_DN, slice_sizes=(1,),
                      mode=lax.GatherScatterMode.PROMISE_IN_BOUNDS)


def _edge_preprocess(src2, dst2, typ2, ea2, le1, e1, le2, e2, n):
    er, w = src2.shape
    br = er
    grid = (er // br,)
    blk = lambda e: (e, 0)

    def body(src_ref, dst_ref, typ_ref, ea_ref, le1_ref, e1_ref,
             le2_ref, e2_ref, iq_ref, ik_ref, ae1_ref, ae2_ref):
        typ = typ_ref[...]
        iq_ref[...] = typ * n + dst_ref[...]
        ik_ref[...] = typ * n + src_ref[...]
        ea = ea_ref[...]
        c1 = jnp.sum(le1_ref[...] * e1_ref[...])
        c2 = jnp.sum(le2_ref[...] * e2_ref[...])
        ae1_ref[...] = ea * c1
        ae2_ref[...] = ea * c2

    return pl.pallas_call(
        body,
        grid=grid,
        in_specs=[
            pl.BlockSpec((br, w), blk),
            pl.BlockSpec((br, w), blk),
            pl.BlockSpec((br, w), blk),
            pl.BlockSpec((br, w), blk),
            pl.BlockSpec((1, F), lambda e: (0, 0)),
            pl.BlockSpec((1, F), lambda e: (0, 0)),
            pl.BlockSpec((1, F), lambda e: (0, 0)),
            pl.BlockSpec((1, F), lambda e: (0, 0)),
        ],
        out_specs=[
            pl.BlockSpec((br, w), blk),
            pl.BlockSpec((br, w), blk),
            pl.BlockSpec((br, w), blk),
            pl.BlockSpec((br, w), blk),
        ],
        out_shape=[
            jax.ShapeDtypeStruct((er, w), jnp.int32),
            jax.ShapeDtypeStruct((er, w), jnp.int32),
            jax.ShapeDtypeStruct((er, w), jnp.float32),
            jax.ShapeDtypeStruct((er, w), jnp.float32),
        ],
    )(src2, dst2, typ2, ea2, le1, e1, le2, e2)


def _dense1_body(x_ref, w_ref, q_ref, k_ref, h_ref, qn_ref, kn_ref):
    h = lax.dot_general(x_ref[...], w_ref[0], (((1,), (0,)), ((), ())),
                        precision=lax.Precision.HIGHEST,
                        preferred_element_type=jnp.float32)
    h_ref[...] = h
    bn = h.shape[0]
    qn_ref[...] = jnp.broadcast_to(
        jnp.sum(h * q_ref[...], axis=1, keepdims=True), (bn, L))
    kn_ref[...] = jnp.broadcast_to(
        jnp.sum(h * k_ref[...], axis=1, keepdims=True), (bn, L))


def _dense1(x, w, qt, kt):
    n = x.shape[0]
    bn = 1000 if n % 1000 == 0 else n
    nb = n // bn
    return pl.pallas_call(
        _dense1_body,
        grid=(R, nb),
        in_specs=[
            pl.BlockSpec((bn, F), lambda r, b: (b, 0)),
            pl.BlockSpec((1, F, F), lambda r, b: (r, 0, 0)),
            pl.BlockSpec((1, F), lambda r, b: (0, 0)),
            pl.BlockSpec((1, F), lambda r, b: (0, 0)),
        ],
        out_specs=[
            pl.BlockSpec((bn, F), lambda r, b: (r * nb + b, 0)),
            pl.BlockSpec((bn, L), lambda r, b: (r * nb + b, 0)),
            pl.BlockSpec((bn, L), lambda r, b: (r * nb + b, 0)),
        ],
        out_shape=[
            jax.ShapeDtypeStruct((R * n, F), jnp.float32),
            jax.ShapeDtypeStruct((R * n, L), jnp.float32),
            jax.ShapeDtypeStruct((R * n, L), jnp.float32),
        ],
    )(x, w, qt, kt)


def _dense2_body(s_ref, d_ref, b_ref, w_ref, q_ref, k_ref,
                 h_ref, qn_ref, kn_ref):
    s = s_ref[0] + s_ref[1]
    d = d_ref[0, :, 0:1] + d_ref[1, :, 0:1]
    hin = jnp.maximum(s / (d + EPS) + b_ref[...], 0.0)
    h = lax.dot_general(hin, w_ref[0], (((1,), (0,)), ((), ())),
                        precision=lax.Precision.HIGHEST,
                        preferred_element_type=jnp.float32)
    h_ref[...] = h
    bn = h.shape[0]
    qn_ref[...] = jnp.broadcast_to(
        jnp.sum(h * q_ref[...], axis=1, keepdims=True), (bn, L))
    kn_ref[...] = jnp.broadcast_to(
        jnp.sum(h * k_ref[...], axis=1, keepdims=True), (bn, L))


def _dense2(s, d, bvec, w, qt, kt):
    n = s.shape[1]
    bn = 1000 if n % 1000 == 0 else n
    nb = n // bn
    return pl.pallas_call(
        _dense2_body,
        grid=(R, nb),
        in_specs=[
            pl.BlockSpec((NC, bn, F), lambda r, b: (0, b, 0)),
            pl.BlockSpec((NC, bn, L), lambda r, b: (0, b, 0)),
            pl.BlockSpec((1, F), lambda r, b: (0, 0)),
            pl.BlockSpec((1, F, F), lambda r, b: (r, 0, 0)),
            pl.BlockSpec((1, F), lambda r, b: (0, 0)),
            pl.BlockSpec((1, F), lambda r, b: (0, 0)),
        ],
        out_specs=[
            pl.BlockSpec((bn, F), lambda r, b: (r * nb + b, 0)),
            pl.BlockSpec((bn, L), lambda r, b: (r * nb + b, 0)),
            pl.BlockSpec((bn, L), lambda r, b: (r * nb + b, 0)),
        ],
        out_shape=[
            jax.ShapeDtypeStruct((R * n, F), jnp.float32),
            jax.ShapeDtypeStruct((R * n, L), jnp.float32),
            jax.ShapeDtypeStruct((R * n, L), jnp.float32),
        ],
    )(s, d, bvec, w, qt, kt)


def _finish_body(s_ref, d_ref, b_ref, out_ref):
    s = s_ref[0] + s_ref[1]
    d = d_ref[0, :, 0:1] + d_ref[1, :, 0:1]
    out_ref[...] = s / (d + EPS) + b_ref[...]


def _finish(s, d, bvec):
    n = s.shape[1]
    bn = 1000 if n % 1000 == 0 else n
    nb = n // bn
    return pl.pallas_call(
        _finish_body,
        grid=(nb,),
        in_specs=[
            pl.BlockSpec((NC, bn, F), lambda b: (0, b, 0)),
            pl.BlockSpec((NC, bn, L), lambda b: (0, b, 0)),
            pl.BlockSpec((1, F), lambda b: (0, 0)),
        ],
        out_specs=pl.BlockSpec((bn, F), lambda b: (b, 0)),
        out_shape=jax.ShapeDtypeStruct((n, F), jnp.float32),
    )(s, d, bvec)


def _make_sc_layer(n, e, interpret=False):
    ew = e // NW
    assert ew * NW == e
    ch = 80
    while ew % ch:
        ch -= 16
    nchunk = ew // ch
    assert nchunk % 2 == 1 and nchunk >= 3
    npairs = (nchunk - 1) // 2
    nzc = n // ch
    assert nzc * ch == n
    zrounds = (nzc + NS - 1) // NS

    mesh = plsc.VectorSubcoreMesh(core_axis_name="c", subcore_axis_name="s",
                                  num_cores=NC, num_subcores=NS)

    @functools.partial(
        pl.kernel,
        out_type=[
            jax.ShapeDtypeStruct((NC, n, F), jnp.float32),
            jax.ShapeDtypeStruct((NC, n, L), jnp.float32),
        ],
        mesh=mesh,
        interpret=interpret,
        compiler_params=pltpu.CompilerParams(needs_layout_passes=False,
                                             use_tc_tiling_on_sc=False),
        scratch_types=[
            pltpu.VMEM((ch, L), jnp.float32),
            pltpu.VMEM((ch, L), jnp.float32),
            pltpu.VMEM((ch, L), jnp.float32),
            pltpu.VMEM((ch, L), jnp.float32),
            pltpu.VMEM((ch, F), jnp.float32),
            pltpu.VMEM((ch, F), jnp.float32),
            pltpu.VMEM((3 * ch,), jnp.int32),
            pltpu.VMEM((3 * ch,), jnp.int32),
            pltpu.VMEM((ch, L), jnp.float32),
            pltpu.VMEM((ch, L), jnp.float32),
            pltpu.VMEM((ch,), jnp.int32),
            pltpu.VMEM((ch,), jnp.int32),
            pltpu.VMEM_SHARED((n, F), jnp.float32),
            pltpu.VMEM_SHARED((n, L), jnp.float32),
            pltpu.SemaphoreType.DMA,
            pltpu.SemaphoreType.DMA,
            pltpu.SemaphoreType.DMA,
            pltpu.SemaphoreType.DMA,
            pltpu.SemaphoreType.DMA,
            pltpu.SemaphoreType.DMA,
            pltpu.SemaphoreType.DMA,
            pltpu.SemaphoreType.DMA,
            pltpu.SemaphoreType.DMA,
            pltpu.SemaphoreType.DMA,
        ],
    )
    def sc_layer(h_hbm, qn_hbm, kn_hbm, pk_hbm, dst_hbm,
                 s_out, d_out, qvA, qvB, kvA, kvB, rowsA, rowsB,
                 pbufA, pbufB, exrowA, exrowB, dstA, dstB, s_sh, d_sh,
                 gsA, qsA, ksA, gsB, qsB, ksB, ssA, dsA, ssB, dsB):
        cid = lax.axis_index("c")
        sid = lax.axis_index("s")
        wid = cid * NS + sid

        zf = jnp.zeros((L,), jnp.float32)

        def zrows(i, carry):
            for j in range(F // L):
                rowsA[i, pl.ds(j * L, L)] = zf
            exrowA[i, :] = zf
            return carry

        lax.fori_loop(0, ch, zrows, 0)

        def zchunk(m, carry):
            c = sid + NS * m

            @pl.when(c < nzc)
            def _():
                pltpu.sync_copy(rowsA, s_sh.at[pl.ds(c * ch, ch)])
                pltpu.sync_copy(exrowA, d_sh.at[pl.ds(c * ch, ch)])

            return carry

        lax.fori_loop(0, zrounds, zchunk, 0)
        plsc.subcore_barrier()

        lane = lax.iota(jnp.int32, 16)
        z16 = jnp.zeros((16,), jnp.int32)

        def pk_load(t, pbuf):
            g = wid * nchunk + t
            pltpu.sync_copy(pk_hbm.at[pl.ds(g * 3 * ch, 3 * ch)], pbuf)

        def dst_load(t, dst_x):
            base = wid * ew + t * ch
            pltpu.sync_copy(dst_hbm.at[pl.ds(base, ch)], dst_x)

        def gather_copies(pbuf, rows_x, qv_x, kv_x, gs, qs, ks):
            iq_r = pbuf.at[pl.ds(0, ch)]
            ik_r = pbuf.at[pl.ds(ch, ch)]
            return (
                pltpu.make_async_copy(h_hbm.at[ik_r], rows_x, gs),
                pltpu.make_async_copy(qn_hbm.at[iq_r], qv_x, qs),
                pltpu.make_async_copy(kn_hbm.at[ik_r], kv_x, ks),
            )

        def issue(bufs):
            for c in gather_copies(*bufs):
                c.start()

        def wait(bufs):
            for c in gather_copies(*bufs):
                c.wait()

        def scatter_copies(rows_x, exrow_x, dst_x, ss, dsm):
            return (
                pltpu.make_async_copy(rows_x, s_sh.at[dst_x], ss),
                pltpu.make_async_copy(exrow_x, d_sh.at[dst_x], dsm),
            )

        def compute(pbuf, rows_x, qv_x, kv_x, exrow_x, dst_x, ss, dsm):
            for g5 in range(ch // 16):
                b16 = g5 * 16
                idx = lane + b16
                q16 = plsc.load_gather(qv_x, [idx, z16])
                k16 = plsc.load_gather(kv_x, [idx, z16])
                a16 = plsc.bitcast(pbuf[pl.ds(2 * ch + b16, 16)], jnp.float32)
                al = q16 + k16 + a16
                al = jnp.maximum(al, NEG_SLOPE * al)
                ex16 = jnp.exp(al)
                for ii in range(16):
                    i = b16 + ii
                    sp = _splat_lane(ex16, z16 + ii)
                    exrow_x[i, :] = sp
                    for j in range(F // L):
                        rows_x[i, pl.ds(j * L, L)] = (
                            rows_x[i, pl.ds(j * L, L)] * sp)

            pltpu.async_copy(rows_x, s_sh.at[dst_x], ss, add=True)
            pltpu.async_copy(exrow_x, d_sh.at[dst_x], dsm, add=True)

        bufsA = (pbufA, rowsA, qvA, kvA, gsA, qsA, ksA)
        bufsB = (pbufB, rowsB, qvB, kvB, gsB, qsB, ksB)
        scatA = (rowsA, exrowA, dstA, ssA, dsA)
        scatB = (rowsB, exrowB, dstB, ssB, dsB)
        compA = (pbufA, rowsA, qvA, kvA, exrowA, dstA, ssA, dsA)
        compB = (pbufB, rowsB, qvB, kvB, exrowB, dstB, ssB, dsB)

        pk_load(0, pbufA)
        dst_load(0, dstA)
        issue(bufsA)
        pk_load(1, pbufB)

        def pair(p, carry):
            t0 = 2 * p

            @pl.when(t0 > 0)
            def _():
                for c in scatter_copies(*scatB):
                    c.wait()

            dst_load(t0 + 1, dstB)
            issue(bufsB)
            wait(bufsA)
            compute(*compA)
            pk_load(t0 + 2, pbufA)
            for c in scatter_copies(*scatA):
                c.wait()
            dst_load(t0 + 2, dstA)
            issue(bufsA)
            wait(bufsB)
            compute(*compB)

            @pl.when(t0 + 3 < nchunk)
            def _():
                pk_load(t0 + 3, pbufB)

            return carry

        lax.fori_loop(0, npairs, pair, 0)
        for c in scatter_copies(*scatB):
            c.wait()
        wait(bufsA)
        compute(*compA)
        for c in scatter_copies(*scatA):
            c.wait()
        plsc.subcore_barrier()

        def ochunk(m, carry):
            c = sid + NS * m

            @pl.when(c < nzc)
            def _():
                pltpu.sync_copy(s_sh.at[pl.ds(c * ch, ch)],
                                s_out.at[cid].at[pl.ds(c * ch, ch)])
                pltpu.sync_copy(d_sh.at[pl.ds(c * ch, ch)],
                                d_out.at[cid].at[pl.ds(c * ch, ch)])

            return carry

        lax.fori_loop(0, zrounds, ochunk, 0)

    return sc_layer


def kernel(x, edge_index, edge_type, edge_attr,
           w1, q1, k1, le1, e1, b1,
           w2, q2, k2, le2, e2, b2):
    n = x.shape[0]
    e = edge_index.shape[1]
    er = e // F
    assert er * F == e

    src = edge_index[0]
    dst = edge_index[1]
    le1t = le1.reshape(1, F)
    e1t = e1.reshape(1, F)
    le2t = le2.reshape(1, F)
    e2t = e2.reshape(1, F)

    iq2, ik2, ae1_2, ae2_2 = _edge_preprocess(
        src.reshape(er, F), dst.reshape(er, F),
        edge_type.reshape(er, F), edge_attr.reshape(er, F),
        le1t, e1t, le2t, e2t, n)
    iq = iq2.reshape(e)
    ik = ik2.reshape(e)
    ae1 = ae1_2.reshape(e)
    ae2 = ae2_2.reshape(e)

    ew = e // NW
    ch = 80
    while ew % ch:
        ch -= 16

    def _pack(ae):
        cols = [iq.reshape(-1, ch), ik.reshape(-1, ch),
                lax.bitcast_convert_type(ae, jnp.int32).reshape(-1, ch)]
        return jnp.stack(cols, axis=1).reshape(-1)

    pk1 = _pack(ae1)
    pk2 = _pack(ae2)

    sc_layer = _make_sc_layer(n, e)

    h1, qn1, kn1 = _dense1(x, w1, q1.reshape(1, F), k1.reshape(1, F))
    s1, d1 = sc_layer(h1, qn1, kn1, pk1, dst)
    h2, qn2, kn2 = _dense2(s1, d1, b1.reshape(1, F), w2,
                           q2.reshape(1, F), k2.reshape(1, F))
    s2, d2 = sc_layer(h2, qn2, kn2, pk2, dst)
    return _finish(s2, d2, b2.reshape(1, F))
